# Initial kernel scaffold; baseline (speedup 1.0000x reference)
import jax
import jax.numpy as jnp
from jax.experimental import pallas as pl


def kernel(x, disc, params):
    raise NotImplementedError("write your pallas kernel here")


# speedup vs baseline: 1.3183x; 1.3183x over previous
import functools

import jax
import jax.numpy as jnp
from jax.experimental import pallas as pl

K = 20
EPS = 1e-5
N = 1024
NEG = -1e30


def _lrelu(t):
    return jnp.where(t >= 0, t, 0.2 * t)


def _norm(t, mean, var, g, b):
    return _lrelu((t - mean) / jnp.sqrt(var + EPS) * g + b)


def _edge_body(xin_ref, w2_ref, emax_ref, ssum_out, ssq_out, idx_ref):
    bidx = pl.program_id(0)
    x = xin_ref[0]

    sq = jnp.sum(x * x, axis=1, keepdims=True)
    gram = jax.lax.dot_general(x, x, (((1,), (1,)), ((), ())),
                               preferred_element_type=jnp.float32)
    pd = 2.0 * gram - sq - jnp.reshape(sq, (1, N))

    w2 = w2_ref[...]
    o = w2.shape[1]
    n_iota = jax.lax.broadcasted_iota(jnp.int32, (1, N), 1)
    k_iota = jax.lax.broadcasted_iota(jnp.int32, (1, K), 1)

    def kadd(s, c, t):
        y = t - c
        ns = s + y
        c = (ns - s) - y
        return ns, c

    def body(t, carry):
        pd, m, s1, c1, sq2, c2, idx = carry
        rmax = jnp.max(pd, axis=1, keepdims=True)
        hot = (pd == rmax).astype(jnp.float32)
        jj = jnp.sum(jnp.where(hot > 0, n_iota, 0), axis=1,
                     keepdims=True)
        idx = jnp.where(k_iota == t, jj, idx)
        xg = jnp.dot(hot, x, precision=jax.lax.Precision.HIGHEST,
                     preferred_element_type=jnp.float32)
        f = jnp.concatenate([xg - x, x], axis=1)
        e = jnp.dot(f, w2, preferred_element_type=jnp.float32)
        m = jnp.maximum(m, e)
        s1, c1 = kadd(s1, c1, e)
        sq2, c2 = kadd(sq2, c2, e * e)
        pd = jnp.where(hot > 0, NEG, pd)
        return pd, m, s1, c1, sq2, c2, idx

    m0 = jnp.full((N, o), NEG, jnp.float32)
    z0 = jnp.zeros((N, o), jnp.float32)
    i0 = jnp.zeros((N, K), jnp.int32)
    pd, m, s1, c1, sq2, c2, idx = jax.lax.fori_loop(
        0, K, body, (pd, m0, z0, z0, z0, z0, i0))

    emax_ref[0] = m
    idx_ref[0] = idx
    psum = jnp.sum(s1 - c1, axis=0, keepdims=True)
    psq = jnp.sum(sq2 - c2, axis=0, keepdims=True)

    @pl.when(bidx == 0)
    def _():
        ssum_out[...] = jnp.zeros_like(ssum_out)
        ssq_out[...] = jnp.zeros_like(ssq_out)

    ssum_out[...] += psum
    ssq_out[...] += psq


def _edge_layer(xin, w2):
    bsz, _, c = xin.shape
    o = w2.shape[1]
    return pl.pallas_call(
        _edge_body,
        grid=(bsz,),
        in_specs=[
            pl.BlockSpec((1, N, c), lambda i: (i, 0, 0)),
            pl.BlockSpec((2 * c, o), lambda i: (0, 0)),
        ],
        out_specs=(
            pl.BlockSpec((1, N, o), lambda i: (i, 0, 0)),
            pl.BlockSpec((1, o), lambda i: (0, 0)),
            pl.BlockSpec((1, o), lambda i: (0, 0)),
            pl.BlockSpec((1, N, K), lambda i: (i, 0, 0)),
        ),
        out_shape=(
            jax.ShapeDtypeStruct((bsz, N, o), jnp.float32),
            jax.ShapeDtypeStruct((1, o), jnp.float32),
            jax.ShapeDtypeStruct((1, o), jnp.float32),
            jax.ShapeDtypeStruct((bsz, N, K), jnp.int32),
        ),
    )(xin, w2)


def _ext_moments(x_bcn, idx, w):
    b, c, n = x_bcn.shape
    idx_base = jnp.arange(b)[:, None, None] * n
    idxf = (idx + idx_base).reshape(-1)
    x_t = jnp.transpose(x_bcn, (0, 2, 1)).reshape(b * n, c)
    feature = x_t[idxf].reshape(b, n, K, c)
    xr = jnp.broadcast_to(jnp.transpose(x_bcn, (0, 2, 1)).reshape(b, n, 1, c),
                          (b, n, K, c))
    feat = jnp.concatenate((feature - xr, xr), axis=3)
    f = jnp.transpose(feat, (0, 3, 1, 2))
    e = jnp.einsum('oi,bink->bonk', w, f)
    return (jnp.mean(e, axis=(0, 2, 3)), jnp.var(e, axis=(0, 2, 3)))


def _cat5_body(x1, x2, x3, x4, w5_ref, y_ref, ssum_out, ssq_out):
    bidx = pl.program_id(0)
    cat = jnp.concatenate([x1[0], x2[0], x3[0], x4[0]], axis=1)
    y = jnp.dot(cat, w5_ref[...], preferred_element_type=jnp.float32)
    y_ref[0] = y

    @pl.when(bidx == 0)
    def _():
        ssum_out[...] = jnp.zeros_like(ssum_out)
        ssq_out[...] = jnp.zeros_like(ssq_out)

    ssum_out[...] += jnp.sum(y, axis=0, keepdims=True)
    ssq_out[...] += jnp.sum(y * y, axis=0, keepdims=True)


def _cat5(xs, w5t, emb):
    bsz = xs[0].shape[0]
    in_specs = [pl.BlockSpec((1, N, xx.shape[2]), lambda i: (i, 0, 0))
                for xx in xs]
    in_specs.append(pl.BlockSpec(w5t.shape, lambda i: (0, 0)))
    return pl.pallas_call(
        _cat5_body,
        grid=(bsz,),
        in_specs=in_specs,
        out_specs=(
            pl.BlockSpec((1, N, emb), lambda i: (i, 0, 0)),
            pl.BlockSpec((1, emb), lambda i: (0, 0)),
            pl.BlockSpec((1, emb), lambda i: (0, 0)),
        ),
        out_shape=(
            jax.ShapeDtypeStruct((bsz, N, emb), jnp.float32),
            jax.ShapeDtypeStruct((1, emb), jnp.float32),
            jax.ShapeDtypeStruct((1, emb), jnp.float32),
        ),
    )(*xs, w5t)


def _pool_body(y_ref, m5, v5, g5, b5, zmax_ref, zavg_ref):
    h = _norm(y_ref[0], m5[...], v5[...], g5[...], b5[...])
    zmax_ref[0] = jnp.max(h, axis=0, keepdims=True)
    zavg_ref[0] = jnp.sum(h, axis=0, keepdims=True) * (1.0 / N)


def _pool(y, m5, v5, g5, b5, emb):
    bsz = y.shape[0]
    stat = pl.BlockSpec((1, emb), lambda i: (0, 0))
    return pl.pallas_call(
        _pool_body,
        grid=(bsz,),
        in_specs=[pl.BlockSpec((1, N, emb), lambda i: (i, 0, 0)),
                  stat, stat, stat, stat],
        out_specs=(pl.BlockSpec((1, 1, emb), lambda i: (i, 0, 0)),
                   pl.BlockSpec((1, 1, emb), lambda i: (i, 0, 0))),
        out_shape=(jax.ShapeDtypeStruct((bsz, 1, emb), jnp.float32),
                   jax.ShapeDtypeStruct((bsz, 1, emb), jnp.float32)),
    )(y, m5, v5, g5, b5)


def _bn_lrelu_rows(t, g, b):
    m = jnp.mean(t, axis=0, keepdims=True)
    var = jnp.mean((t - m) * (t - m), axis=0, keepdims=True)
    return _lrelu((t - m) / jnp.sqrt(var + EPS) * g + b)


def _head_body(z_ref, w1, g6, b6, w2, g7, b7, w21, w22, g8, b8, w3, out_ref):
    z = z_ref[...]
    t = _bn_lrelu_rows(
        jnp.dot(z, w1[...], preferred_element_type=jnp.float32),
        g6[...], b6[...])
    t = _bn_lrelu_rows(
        jnp.dot(t, w2[...], preferred_element_type=jnp.float32),
        g7[...], b7[...])
    t = jnp.dot(t, w21[...], preferred_element_type=jnp.float32)
    t = jnp.dot(t, w22[...], preferred_element_type=jnp.float32)
    m = jnp.mean(t, axis=0, keepdims=True)
    var = jnp.mean((t - m) * (t - m), axis=0, keepdims=True)
    tn = (t - m) / jnp.sqrt(var + EPS) * g8[...] + b8[...]
    ge = 0.5 * tn * (1.0 + jax.lax.erf(tn * (2.0 ** -0.5)))
    out_ref[...] = jnp.dot(ge, w3[...], preferred_element_type=jnp.float32)


def _head(z, ws):
    full = lambda a: pl.BlockSpec(a.shape, None)
    return pl.pallas_call(
        _head_body,
        in_specs=[full(z)] + [full(w) for w in ws],
        out_specs=pl.BlockSpec((z.shape[0], 90), None),
        out_shape=jax.ShapeDtypeStruct((z.shape[0], 90), jnp.float32),
    )(z, *ws)


def kernel(x, disc, params):
    p = params
    bsz = x.shape[0]
    cnt = float(bsz * N * K)
    xt = jnp.transpose(x, (0, 2, 1)).astype(jnp.float32)

    def row(a):
        return jnp.reshape(a, (1, -1)).astype(jnp.float32)

    def glue(emax, m, v, g, b):
        xn = (emax - m[None, None, :]) / jnp.sqrt(v[None, None, :] + EPS)
        return jax.lax.optimization_barrier(
            _lrelu(xn * g[None, None, :] + b[None, None, :]))

    e1, s1, q1, i1 = _edge_layer(xt, jnp.transpose(p['conv1_w']))
    m1, v1 = _ext_moments(x, i1.astype(jnp.int32), p['conv1_w'])
    x1 = glue(e1, m1, v1, p['bn1_g'], p['bn1_b'])

    e2, s2, q2, i2 = _edge_layer(x1, jnp.transpose(p['conv2_w']))
    m2, v2 = _ext_moments(jnp.transpose(x1, (0, 2, 1)), i2.astype(jnp.int32),
                          p['conv2_w'])
    x2 = glue(e2, m2, v2, p['bn2_g'], p['bn2_b'])

    e3, s3, q3, i3 = _edge_layer(x2, jnp.transpose(p['conv3_w']))
    m3, v3 = _ext_moments(jnp.transpose(x2, (0, 2, 1)), i3.astype(jnp.int32),
                          p['conv3_w'])
    x3 = glue(e3, m3, v3, p['bn3_g'], p['bn3_b'])

    e4, s4, q4, i4 = _edge_layer(x3, jnp.transpose(p['conv4_w']))
    m4 = (s4 / cnt).reshape(-1)
    v4 = (q4 / cnt).reshape(-1) - m4 * m4
    x4 = glue(e4, m4, v4, p['bn4_g'], p['bn4_b'])

    emb = p['conv5_w'].shape[0]
    y, s5, q5 = _cat5([x1, x2, x3, x4], jnp.transpose(p['conv5_w']), emb)
    cnt5 = float(bsz * N)
    m5 = s5 / cnt5
    v5 = q5 / cnt5 - m5 * m5
    zmax, zavg = _pool(y, m5, v5, row(p['bn5_g']), row(p['bn5_b']), emb)
    z = jnp.concatenate([zmax[:, 0, :], zavg[:, 0, :]], axis=1)

    ws = [jnp.transpose(p['lin1_w']), row(p['bn6_g']), row(p['bn6_b']),
          jnp.transpose(p['lin2_w']), row(p['bn7_g']), row(p['bn7_b']),
          jnp.transpose(p['lin21_w']), jnp.transpose(p['lin22_w']),
          row(p['bn8_g']), row(p['bn8_b']), jnp.transpose(p['lin3_w'])]
    return _head(z, ws)
